# Initial kernel scaffold; baseline (speedup 1.0000x reference)
import jax
import jax.numpy as jnp
from jax.experimental import pallas as pl


def kernel(start_day, end_day, adj_indices, adj_values, edges, W0, b0, W1, b1, W_ih, W_hh, b_ih, b_hh, gamma, beta, Wp1, bp1, Wp2, bp2):
    raise NotImplementedError("write your pallas kernel here")


# speedup vs baseline: 2.6333x; 2.6333x over previous
import functools

import jax
import jax.numpy as jnp
from jax import lax
from jax.experimental import pallas as pl
from jax.experimental.pallas import tpu as pltpu
from jax.experimental.pallas import tpu_sc as plsc

_N = 50000
_T = 3
_E = 800000
_D = 64
_H = 64
_NHE = 128
_EP = 100000

_NC = 2
_NS = 16
_HALF = _N // _NC
_CH = 384
_NCHUNK = 131
_EPT = _CH * _NCHUNK
_EPAD = _NS * _EPT
_WSLICE = 1568

_EPP = 3200
_EPPAD = _EPP * _NC * _NS

_mesh = functools.partial(
    plsc.VectorSubcoreMesh, core_axis_name="c", subcore_axis_name="s",
    num_cores=_NC, num_subcores=_NS)


def _spmm_body(rows_hbm, cols_hbm, vals_hbm, d_hbm, out_hbm,
               rowbuf, colbuf, valbuf, locbuf, gbuf, acc, gsem):
    cid = lax.axis_index("c")
    sid = lax.axis_index("s")
    lo = cid * _HALF

    zero = jnp.zeros((16,), jnp.float32)

    def _zrow(i, carry):
        for q in range(4):
            gbuf[i, pl.ds(q * 16, 16)] = zero
        return carry

    lax.fori_loop(0, _CH, _zrow, 0)
    zstart = jnp.minimum(sid * _WSLICE, _HALF - _WSLICE)
    for z in range(4):
        pltpu.sync_copy(gbuf.at[pl.ds(0, _CH), :],
                        acc.at[pl.ds(zstart + z * _CH, _CH), :])
    pltpu.sync_copy(gbuf.at[pl.ds(0, 32), :],
                    acc.at[pl.ds(zstart + 4 * _CH, 32), :])
    plsc.subcore_barrier()

    def _chunk(c, carry):
        base = sid * _EPT + c * _CH
        pltpu.sync_copy(rows_hbm.at[pl.ds(base, _CH)], rowbuf)
        pltpu.sync_copy(cols_hbm.at[pl.ds(base, _CH)], colbuf)
        pltpu.sync_copy(vals_hbm.at[pl.ds(base, _CH)], valbuf)
        descs = [
            pltpu.async_copy(d_hbm.at[colbuf.at[pl.ds(b * 128, 128)]],
                             gbuf.at[pl.ds(b * 128, 128), :], gsem)
            for b in range(_CH // 128)
        ]
        for de in descs:
            de.wait()

        def _grp(j, carry2):
            j16 = j * 16
            dst = rowbuf[pl.ds(j16, 16)]
            val = valbuf[pl.ds(j16, 16)]
            m = (dst >= lo) & (dst < lo + _HALF)
            locbuf[j // 8, pl.ds((j % 8) * 16, 16)] = jnp.where(m, dst - lo, 0)
            valm = jnp.where(m, val, 0.0)
            for k in range(16):
                r = j16 + k
                v = valm[k]
                for q in range(4):
                    gbuf[r, pl.ds(q * 16, 16)] = gbuf[r, pl.ds(q * 16, 16)] * v
            return carry2

        lax.fori_loop(0, _CH // 16, _grp, 0)
        for b in range(_CH // 128):
            pltpu.sync_copy(gbuf.at[pl.ds(b * 128, 128), :],
                            acc.at[locbuf.at[b]], add=True)
        return carry

    lax.fori_loop(0, _NCHUNK, _chunk, 0)
    plsc.subcore_barrier()

    start = jnp.minimum(sid * _WSLICE, _HALF - _WSLICE)
    pltpu.sync_copy(acc.at[pl.ds(start, _WSLICE), :],
                    out_hbm.at[pl.ds(lo + start, _WSLICE), :])


_spmm = pl.kernel(
    _spmm_body,
    out_type=jax.ShapeDtypeStruct((_N, _D), jnp.float32),
    mesh=_mesh(),
    compiler_params=pltpu.CompilerParams(use_tc_tiling_on_sc=False),
    scratch_types=[
        pltpu.VMEM((_CH,), jnp.int32),
        pltpu.VMEM((_CH,), jnp.int32),
        pltpu.VMEM((_CH,), jnp.float32),
        pltpu.VMEM((_CH // 128, 128), jnp.int32),
        pltpu.VMEM((_CH, _D), jnp.float32),
        pltpu.VMEM_SHARED((_HALF, _D), jnp.float32),
        pltpu.SemaphoreType.DMA,
    ],
)


def _edge_body(h_hbm, e_hbm, ac_hbm, feats_hbm,
               ebuf, g0, g1, fbuf, acbuf, gsem):
    cid = lax.axis_index("c")
    sid = lax.axis_index("s")
    w = sid * _NC + cid
    base = w * _EPP
    pltpu.sync_copy(ac_hbm, acbuf)
    nu = _EPP // 128
    descs = []
    for u in range(nu):
        descs.append(pltpu.async_copy(
            e_hbm.at[0, pl.ds(base + u * 128, 128)], ebuf.at[0, u], gsem))
        descs.append(pltpu.async_copy(
            e_hbm.at[1, pl.ds(base + u * 128, 128)], ebuf.at[1, u], gsem))
    for de in descs:
        de.wait()

    def _unit(u, carry):
        da = pltpu.async_copy(h_hbm.at[ebuf.at[0, u]], g0, gsem)
        db = pltpu.async_copy(h_hbm.at[ebuf.at[1, u]], g1, gsem)
        da.wait()
        db.wait()

        def _row(j, carry2):
            for q in range(4):
                a_q = acbuf[0, pl.ds(q * 16, 16)]
                c_q = acbuf[1, pl.ds(q * 16, 16)]
                fbuf[j, pl.ds(q * 16, 16)] = g0[j, pl.ds(q * 16, 16)] * a_q + c_q
                fbuf[j, pl.ds(64 + q * 16, 16)] = g1[j, pl.ds(q * 16, 16)] * a_q + c_q
            return carry2

        lax.fori_loop(0, 128, _row, 0)
        pltpu.sync_copy(fbuf, feats_hbm.at[pl.ds(base + u * 128, 128), :])
        return carry

    lax.fori_loop(0, nu, _unit, 0)


_edge = pl.kernel(
    _edge_body,
    out_type=jax.ShapeDtypeStruct((_EPPAD, 2 * _H), jnp.float32),
    mesh=_mesh(),
    compiler_params=pltpu.CompilerParams(use_tc_tiling_on_sc=False),
    scratch_types=[
        pltpu.VMEM((2, _EPP // 128, 128), jnp.int32),
        pltpu.VMEM((128, _H), jnp.float32),
        pltpu.VMEM((128, _H), jnp.float32),
        pltpu.VMEM((128, 2 * _H), jnp.float32),
        pltpu.VMEM((2, _H), jnp.float32),
        pltpu.SemaphoreType.DMA,
    ],
)


_BLK = 2000


def _sup_body(x_ref, b0_ref, w1_ref, o_ref):
    x = jnp.maximum(x_ref[...] + b0_ref[...], 0.0)
    o_ref[...] = jnp.dot(x, w1_ref[...], preferred_element_type=jnp.float32)


def _sup(x, b0, W1):
    grid = _N // _BLK
    return pl.pallas_call(
        _sup_body,
        grid=(grid,),
        in_specs=[
            pl.BlockSpec((_BLK, _D), lambda i: (i, 0)),
            pl.BlockSpec((1, _D), lambda i: (0, 0)),
            pl.BlockSpec((_D, _D), lambda i: (0, 0)),
        ],
        out_specs=pl.BlockSpec((_BLK, _D), lambda i: (i, 0)),
        out_shape=jax.ShapeDtypeStruct((_N, _D), jnp.float32),
    )(x, b0.reshape(1, _D), W1)


def _gru_body(x0_ref, x1_ref, x2_ref, wihT_ref, whhT_ref, bih_ref, bhh_ref,
              b1_ref, tm_ref, h_ref, st_ref):
    i = pl.program_id(0)
    h = jnp.zeros((_BLK, _H), jnp.float32)
    xs = (x0_ref, x1_ref, x2_ref)
    wihT = wihT_ref[...]
    whhT = whhT_ref[...]
    for t in range(_T):
        x = xs[t][...] + b1_ref[...]
        gi = jnp.dot(x, wihT, preferred_element_type=jnp.float32) + bih_ref[...]
        gh = jnp.dot(h, whhT, preferred_element_type=jnp.float32) + bhh_ref[...]
        r = jax.nn.sigmoid(gi[:, :_H] + gh[:, :_H])
        z = jax.nn.sigmoid(gi[:, _H:2 * _H] + gh[:, _H:2 * _H])
        n = jnp.tanh(gi[:, 2 * _H:] + r * gh[:, 2 * _H:])
        h_new = (1.0 - z) * n + z * h
        tm = tm_ref[0, t]
        h = tm * h_new + (1.0 - tm) * h
    h_ref[...] = h

    @pl.when(i == 0)
    def _():
        st_ref[...] = jnp.zeros_like(st_ref)

    st_ref[0:1, :] += jnp.sum(h, axis=0, keepdims=True)
    st_ref[1:2, :] += jnp.sum(h * h, axis=0, keepdims=True)


def _gru(x0, x1, x2, wihT, whhT, b_ih, b_hh, b1, tmask):
    grid = _N // _BLK
    return pl.pallas_call(
        _gru_body,
        grid=(grid,),
        in_specs=[
            pl.BlockSpec((_BLK, _D), lambda i: (i, 0)),
            pl.BlockSpec((_BLK, _D), lambda i: (i, 0)),
            pl.BlockSpec((_BLK, _D), lambda i: (i, 0)),
            pl.BlockSpec((_D, 3 * _H), lambda i: (0, 0)),
            pl.BlockSpec((_H, 3 * _H), lambda i: (0, 0)),
            pl.BlockSpec((1, 3 * _H), lambda i: (0, 0)),
            pl.BlockSpec((1, 3 * _H), lambda i: (0, 0)),
            pl.BlockSpec((1, _D), lambda i: (0, 0)),
            pl.BlockSpec((1, _T), lambda i: (0, 0), memory_space=pltpu.SMEM),
        ],
        out_specs=[
            pl.BlockSpec((_BLK, _H), lambda i: (i, 0)),
            pl.BlockSpec((8, _H), lambda i: (0, 0)),
        ],
        out_shape=[
            jax.ShapeDtypeStruct((_N, _H), jnp.float32),
            jax.ShapeDtypeStruct((8, _H), jnp.float32),
        ],
    )(x0, x1, x2, wihT, whhT, b_ih.reshape(1, -1), b_hh.reshape(1, -1),
      b1.reshape(1, -1), tmask)


_MBLK = 2048


def _mlp_body(f_ref, wp1_ref, bp1_ref, wp2_ref, bp2_ref, o_ref):
    hmid = jnp.maximum(
        jnp.dot(f_ref[...], wp1_ref[...], preferred_element_type=jnp.float32)
        + bp1_ref[...], 0.0)
    lg = jnp.dot(hmid, wp2_ref[...], preferred_element_type=jnp.float32) \
        + bp2_ref[...]
    m = jnp.max(lg, axis=1, keepdims=True)
    e = jnp.exp(lg - m)
    o_ref[...] = (lg - m) - jnp.log(jnp.sum(e, axis=1, keepdims=True))


def _mlp(feats, Wp1, bp1, Wp2, bp2):
    grid = _EPPAD // _MBLK
    return pl.pallas_call(
        _mlp_body,
        grid=(grid,),
        in_specs=[
            pl.BlockSpec((_MBLK, _NHE), lambda i: (i, 0)),
            pl.BlockSpec((_NHE, _NHE), lambda i: (0, 0)),
            pl.BlockSpec((1, _NHE), lambda i: (0, 0)),
            pl.BlockSpec((_NHE, 2), lambda i: (0, 0)),
            pl.BlockSpec((1, 2), lambda i: (0, 0)),
        ],
        out_specs=pl.BlockSpec((_MBLK, 2), lambda i: (i, 0)),
        out_shape=jax.ShapeDtypeStruct((_EPPAD, 2), jnp.float32),
    )(feats, Wp1, bp1.reshape(1, -1), Wp2, bp2.reshape(1, -1))


def kernel(start_day, end_day, adj_indices, adj_values, edges,
           W0, b0, W1, b1, W_ih, W_hh, b_ih, b_hh, gamma, beta,
           Wp1, bp1, Wp2, bp2):
    adj_indices = adj_indices.astype(jnp.int32)
    edges = edges.astype(jnp.int32)
    pad = _EPAD - _E
    adi = jnp.pad(adj_indices, ((0, 0), (0, 0), (0, pad)))
    adv = jnp.pad(adj_values, ((0, 0), (0, pad)))

    outs = []
    for i in range(_T):
        t = start_day + i
        idx_t = lax.dynamic_index_in_dim(adi, t, 0, keepdims=False)
        val_t = lax.dynamic_index_in_dim(adv, t, 0, keepdims=False)
        x1 = _spmm(idx_t[0], idx_t[1], val_t, W0)
        sup = _sup(x1, b0, W1)
        outs.append(_spmm(idx_t[0], idx_t[1], val_t, sup))

    tmask = ((start_day + jnp.arange(_T)) <= end_day) \
        .astype(jnp.float32).reshape(1, _T)
    h, stats = _gru(outs[0], outs[1], outs[2], W_ih.T, W_hh.T,
                    b_ih, b_hh, b1, tmask)
    mean = stats[0, :] / _N
    var = stats[1, :] / _N - mean * mean
    a = gamma * lax.rsqrt(var + 1e-5)
    c = beta - a * mean

    epad = jnp.pad(edges, ((0, 0), (0, _EPPAD - _EP)))
    feats = _edge(h, epad, jnp.stack([a, c]))
    return _mlp(feats, Wp1, bp1, Wp2, bp2)[:_EP]


# speedup vs baseline: 6.7205x; 2.5521x over previous
import functools

import jax
import jax.numpy as jnp
from jax import lax
from jax.experimental import pallas as pl
from jax.experimental.pallas import tpu as pltpu
from jax.experimental.pallas import tpu_sc as plsc

_N = 50000
_T = 3
_E = 800000
_D = 64
_H = 64
_NHE = 128
_EP = 100000

_NC = 2
_NS = 16
_DH = _D // _NC
_U = 128
_UPB = 23
_NBLK = 17
_UPT = _UPB * _NBLK
_EPT = _UPT * _U
_EPAD = _NS * _EPT
_EU = _EPAD // _U
_WSLICE = 3136
_WLAST = _N - _WSLICE

_EPP = 3200
_EPPAD = _EPP * _NC * _NS

_mesh = functools.partial(
    plsc.VectorSubcoreMesh, core_axis_name="c", subcore_axis_name="s",
    num_cores=_NC, num_subcores=_NS)


def _spmm_body(rows_hbm, cols_hbm, vals_hbm, d_hbm, out_hbm,
               rowbuf, colbuf, valbuf, gsub, acc, gsem, ssem, zsem):
    cid = lax.axis_index("c")
    sid = lax.axis_index("s")
    dloc = d_hbm.at[cid]

    zero = jnp.zeros((16,), jnp.float32)

    def _zrow(i, carry):
        for q in range(_DH // 16):
            gsub[0, i, pl.ds(q * 16, 16)] = zero
        return carry

    lax.fori_loop(0, _U, _zrow, 0)
    zstart = jnp.minimum(sid * _WSLICE, _WLAST)
    zdescs = [
        pltpu.async_copy(gsub.at[0],
                         acc.at[pl.ds(zstart + z * _U, _U), :], zsem)
        for z in range(24)
    ]
    zdescs.append(pltpu.async_copy(
        gsub.at[0, pl.ds(0, 64), :],
        acc.at[pl.ds(zstart + 24 * _U, 64), :], zsem))
    for de in zdescs:
        de.wait()
    plsc.subcore_barrier()

    ubase = sid * _UPT
    for b in range(_NBLK):
        boff = ubase + b * _UPB
        di = pltpu.async_copy(rows_hbm.at[pl.ds(boff, _UPB), :], rowbuf, gsem)
        dj = pltpu.async_copy(cols_hbm.at[pl.ds(boff, _UPB), :], colbuf, gsem)
        dk = pltpu.async_copy(vals_hbm.at[pl.ds(boff, _UPB), :], valbuf, gsem)
        di.wait()
        dj.wait()
        dk.wait()
        pltpu.async_copy(dloc.at[colbuf.at[0]], gsub.at[0], gsem)

        def _unit(u, carry):
            p = u % 2
            pn = (u + 1) % 2

            @pl.when(u >= 1)
            def _():
                pltpu.make_async_copy(dloc.at[pl.ds(0, _U)],
                                      gsub.at[pn], ssem).wait()

            @pl.when(u + 1 < _UPB)
            def _():
                pltpu.async_copy(dloc.at[colbuf.at[u + 1]], gsub.at[pn], gsem)

            pltpu.make_async_copy(dloc.at[pl.ds(0, _U)], gsub.at[p], gsem).wait()

            def _grp(g, carry2):
                g16 = g * 16
                val = valbuf[u, pl.ds(g16, 16)]
                for k in range(16):
                    r = g16 + k
                    v = val[k]
                    for q in range(_DH // 16):
                        gsub[p, r, pl.ds(q * 16, 16)] = \
                            gsub[p, r, pl.ds(q * 16, 16)] * v
                return carry2

            lax.fori_loop(0, _U // 16, _grp, 0)

            pltpu.async_copy(gsub.at[p], acc.at[rowbuf.at[u]], ssem, add=True)
            return carry

        lax.fori_loop(0, _UPB, _unit, 0)
        pltpu.make_async_copy(dloc.at[pl.ds(0, _U)],
                              gsub.at[(_UPB - 1) % 2], ssem).wait()

    plsc.subcore_barrier()
    start = jnp.minimum(sid * _WSLICE, _WLAST)
    pltpu.sync_copy(acc.at[pl.ds(start, _WSLICE), :],
                    out_hbm.at[cid, pl.ds(start, _WSLICE), :])


_spmm = pl.kernel(
    _spmm_body,
    out_type=jax.ShapeDtypeStruct((_NC, _N, _DH), jnp.float32),
    mesh=_mesh(),
    compiler_params=pltpu.CompilerParams(use_tc_tiling_on_sc=False),
    scratch_types=[
        pltpu.VMEM((_UPB, _U), jnp.int32),
        pltpu.VMEM((_UPB, _U), jnp.int32),
        pltpu.VMEM((_UPB, _U), jnp.float32),
        pltpu.VMEM((2, _U, _DH), jnp.float32),
        pltpu.VMEM_SHARED((_N, _DH), jnp.float32),
        pltpu.SemaphoreType.DMA,
        pltpu.SemaphoreType.DMA,
        pltpu.SemaphoreType.DMA,
    ],
)


def _edge_body(h_hbm, e_hbm, ac_hbm, feats_hbm,
               ebuf, g0, g1, fbuf, acbuf, gsem):
    cid = lax.axis_index("c")
    sid = lax.axis_index("s")
    w = sid * _NC + cid
    base = w * _EPP
    pltpu.sync_copy(ac_hbm, acbuf)
    nu = _EPP // 128
    descs = []
    for u in range(nu):
        descs.append(pltpu.async_copy(
            e_hbm.at[0, pl.ds(base + u * 128, 128)], ebuf.at[0, u], gsem))
        descs.append(pltpu.async_copy(
            e_hbm.at[1, pl.ds(base + u * 128, 128)], ebuf.at[1, u], gsem))
    for de in descs:
        de.wait()

    def _unit(u, carry):
        da = pltpu.async_copy(h_hbm.at[ebuf.at[0, u]], g0, gsem)
        db = pltpu.async_copy(h_hbm.at[ebuf.at[1, u]], g1, gsem)
        da.wait()
        db.wait()

        def _row(j, carry2):
            for q in range(4):
                a_q = acbuf[0, pl.ds(q * 16, 16)]
                c_q = acbuf[1, pl.ds(q * 16, 16)]
                fbuf[j, pl.ds(q * 16, 16)] = g0[j, pl.ds(q * 16, 16)] * a_q + c_q
                fbuf[j, pl.ds(64 + q * 16, 16)] = g1[j, pl.ds(q * 16, 16)] * a_q + c_q
            return carry2

        lax.fori_loop(0, 128, _row, 0)
        pltpu.sync_copy(fbuf, feats_hbm.at[pl.ds(base + u * 128, 128), :])
        return carry

    lax.fori_loop(0, nu, _unit, 0)


_edge = pl.kernel(
    _edge_body,
    out_type=jax.ShapeDtypeStruct((_EPPAD, 2 * _H), jnp.float32),
    mesh=_mesh(),
    compiler_params=pltpu.CompilerParams(use_tc_tiling_on_sc=False),
    scratch_types=[
        pltpu.VMEM((2, _EPP // 128, 128), jnp.int32),
        pltpu.VMEM((128, _H), jnp.float32),
        pltpu.VMEM((128, _H), jnp.float32),
        pltpu.VMEM((128, 2 * _H), jnp.float32),
        pltpu.VMEM((2, _H), jnp.float32),
        pltpu.SemaphoreType.DMA,
    ],
)


_BLK = 2000


def _sup_body(x_ref, b0_ref, w1_ref, o_ref):
    x = jnp.concatenate([x_ref[0], x_ref[1]], axis=1)
    x = jnp.maximum(x + b0_ref[...], 0.0)
    res = jnp.dot(x, w1_ref[...], preferred_element_type=jnp.float32)
    o_ref[0] = res[:, :_DH]
    o_ref[1] = res[:, _DH:]


def _sup(x, b0, W1):
    grid = _N // _BLK
    return pl.pallas_call(
        _sup_body,
        grid=(grid,),
        in_specs=[
            pl.BlockSpec((_NC, _BLK, _DH), lambda i: (0, i, 0)),
            pl.BlockSpec((1, _D), lambda i: (0, 0)),
            pl.BlockSpec((_D, _D), lambda i: (0, 0)),
        ],
        out_specs=pl.BlockSpec((_NC, _BLK, _DH), lambda i: (0, i, 0)),
        out_shape=jax.ShapeDtypeStruct((_NC, _N, _DH), jnp.float32),
    )(x, b0.reshape(1, _D), W1)


def _gru_body(x0_ref, x1_ref, x2_ref, wihT_ref, whhT_ref, bih_ref, bhh_ref,
              b1_ref, tm_ref, h_ref, st_ref):
    i = pl.program_id(0)
    h = jnp.zeros((_BLK, _H), jnp.float32)
    xs = (x0_ref, x1_ref, x2_ref)
    wihT = wihT_ref[...]
    whhT = whhT_ref[...]
    for t in range(_T):
        x = jnp.concatenate([xs[t][0], xs[t][1]], axis=1) + b1_ref[...]
        gi = jnp.dot(x, wihT, preferred_element_type=jnp.float32) + bih_ref[...]
        gh = jnp.dot(h, whhT, preferred_element_type=jnp.float32) + bhh_ref[...]
        r = jax.nn.sigmoid(gi[:, :_H] + gh[:, :_H])
        z = jax.nn.sigmoid(gi[:, _H:2 * _H] + gh[:, _H:2 * _H])
        n = jnp.tanh(gi[:, 2 * _H:] + r * gh[:, 2 * _H:])
        h_new = (1.0 - z) * n + z * h
        tm = tm_ref[0, t]
        h = tm * h_new + (1.0 - tm) * h
    h_ref[...] = h

    @pl.when(i == 0)
    def _():
        st_ref[...] = jnp.zeros_like(st_ref)

    st_ref[0:1, :] += jnp.sum(h, axis=0, keepdims=True)
    st_ref[1:2, :] += jnp.sum(h * h, axis=0, keepdims=True)


def _gru(x0, x1, x2, wihT, whhT, b_ih, b_hh, b1, tmask):
    grid = _N // _BLK
    return pl.pallas_call(
        _gru_body,
        grid=(grid,),
        in_specs=[
            pl.BlockSpec((_NC, _BLK, _DH), lambda i: (0, i, 0)),
            pl.BlockSpec((_NC, _BLK, _DH), lambda i: (0, i, 0)),
            pl.BlockSpec((_NC, _BLK, _DH), lambda i: (0, i, 0)),
            pl.BlockSpec((_D, 3 * _H), lambda i: (0, 0)),
            pl.BlockSpec((_H, 3 * _H), lambda i: (0, 0)),
            pl.BlockSpec((1, 3 * _H), lambda i: (0, 0)),
            pl.BlockSpec((1, 3 * _H), lambda i: (0, 0)),
            pl.BlockSpec((1, _D), lambda i: (0, 0)),
            pl.BlockSpec((1, _T), lambda i: (0, 0), memory_space=pltpu.SMEM),
        ],
        out_specs=[
            pl.BlockSpec((_BLK, _H), lambda i: (i, 0)),
            pl.BlockSpec((8, _H), lambda i: (0, 0)),
        ],
        out_shape=[
            jax.ShapeDtypeStruct((_N, _H), jnp.float32),
            jax.ShapeDtypeStruct((8, _H), jnp.float32),
        ],
    )(x0, x1, x2, wihT, whhT, b_ih.reshape(1, -1), b_hh.reshape(1, -1),
      b1.reshape(1, -1), tmask)


_MBLK = 2048


def _mlp_body(f_ref, wp1_ref, bp1_ref, wp2_ref, bp2_ref, o_ref):
    hmid = jnp.maximum(
        jnp.dot(f_ref[...], wp1_ref[...], preferred_element_type=jnp.float32)
        + bp1_ref[...], 0.0)
    lg = jnp.dot(hmid, wp2_ref[...], preferred_element_type=jnp.float32) \
        + bp2_ref[...]
    m = jnp.max(lg, axis=1, keepdims=True)
    e = jnp.exp(lg - m)
    o_ref[...] = (lg - m) - jnp.log(jnp.sum(e, axis=1, keepdims=True))


def _mlp(feats, Wp1, bp1, Wp2, bp2):
    grid = _EPPAD // _MBLK
    return pl.pallas_call(
        _mlp_body,
        grid=(grid,),
        in_specs=[
            pl.BlockSpec((_MBLK, _NHE), lambda i: (i, 0)),
            pl.BlockSpec((_NHE, _NHE), lambda i: (0, 0)),
            pl.BlockSpec((1, _NHE), lambda i: (0, 0)),
            pl.BlockSpec((_NHE, 2), lambda i: (0, 0)),
            pl.BlockSpec((1, 2), lambda i: (0, 0)),
        ],
        out_specs=pl.BlockSpec((_MBLK, 2), lambda i: (i, 0)),
        out_shape=jax.ShapeDtypeStruct((_EPPAD, 2), jnp.float32),
    )(feats, Wp1, bp1.reshape(1, -1), Wp2, bp2.reshape(1, -1))


def kernel(start_day, end_day, adj_indices, adj_values, edges,
           W0, b0, W1, b1, W_ih, W_hh, b_ih, b_hh, gamma, beta,
           Wp1, bp1, Wp2, bp2):
    adj_indices = adj_indices.astype(jnp.int32)
    edges = edges.astype(jnp.int32)
    pad = _EPAD - _E
    adi = jnp.pad(adj_indices, ((0, 0), (0, 0), (0, pad)))
    adv = jnp.pad(adj_values, ((0, 0), (0, pad)))
    W0s = jnp.stack([W0[:, :_DH], W0[:, _DH:]])

    outs = []
    for i in range(_T):
        t = start_day + i
        idx_t = lax.dynamic_index_in_dim(adi, t, 0, keepdims=False)
        val_t = lax.dynamic_index_in_dim(adv, t, 0, keepdims=False)
        r2 = idx_t[0].reshape(_EU, _U)
        c2 = idx_t[1].reshape(_EU, _U)
        v2 = val_t.reshape(_EU, _U)
        x1 = _spmm(r2, c2, v2, W0s)
        sup = _sup(x1, b0, W1)
        outs.append(_spmm(r2, c2, v2, sup))

    tmask = ((start_day + jnp.arange(_T)) <= end_day) \
        .astype(jnp.float32).reshape(1, _T)
    h, stats = _gru(outs[0], outs[1], outs[2], W_ih.T, W_hh.T,
                    b_ih, b_hh, b1, tmask)
    mean = stats[0, :] / _N
    var = stats[1, :] / _N - mean * mean
    a = gamma * lax.rsqrt(var + 1e-5)
    c = beta - a * mean

    epad = jnp.pad(edges, ((0, 0), (0, _EPPAD - _EP)))
    feats = _edge(h, epad, jnp.stack([a, c]))
    return _mlp(feats, Wp1, bp1, Wp2, bp2)[:_EP]


# speedup vs baseline: 7.5303x; 1.1205x over previous
import functools

import jax
import jax.numpy as jnp
from jax import lax
from jax.experimental import pallas as pl
from jax.experimental.pallas import tpu as pltpu
from jax.experimental.pallas import tpu_sc as plsc

_N = 50000
_T = 3
_E = 800000
_D = 64
_H = 64
_NHE = 128
_EP = 100000

_NC = 2
_NS = 16
_DH = _D // _NC
_U = 128
_G = 2
_SPB = 14
_UPB = _SPB * _G
_NBLK = 14
_UPT = _UPB * _NBLK
_EPT = _UPT * _U
_EPAD = _NS * _EPT
_EU = _EPAD // _U
_WSLICE = 3136
_WLAST = _N - _WSLICE

_EPP = 3200
_EPPAD = _EPP * _NC * _NS

_mesh = functools.partial(
    plsc.VectorSubcoreMesh, core_axis_name="c", subcore_axis_name="s",
    num_cores=_NC, num_subcores=_NS)


def _spmm_body(rows_hbm, cols_hbm, vals_hbm, d_hbm, out_hbm,
               rowbuf, colbuf, valbuf, gsub, acc, gsem, ssem, zsem):
    cid = lax.axis_index("c")
    sid = lax.axis_index("s")
    dloc = d_hbm.at[cid]

    zero = jnp.zeros((16,), jnp.float32)

    def _zrow(i, carry):
        for q in range(_DH // 16):
            gsub[0, i, pl.ds(q * 16, 16)] = zero
        return carry

    lax.fori_loop(0, _G * _U, _zrow, 0)
    zstart = jnp.minimum(sid * _WSLICE, _WLAST)
    zn = _G * _U
    zdescs = [
        pltpu.async_copy(gsub.at[0],
                         acc.at[pl.ds(zstart + z * zn, zn), :], zsem)
        for z in range(_WSLICE // zn)
    ]
    zdescs.append(pltpu.async_copy(
        gsub.at[0, pl.ds(0, _WSLICE % zn), :],
        acc.at[pl.ds(zstart + (_WSLICE // zn) * zn, _WSLICE % zn), :], zsem))
    for de in zdescs:
        de.wait()
    plsc.subcore_barrier()

    ubase = sid * _UPT
    for b in range(_NBLK):
        boff = ubase + b * _UPB
        di = pltpu.async_copy(rows_hbm.at[pl.ds(boff, _UPB), :], rowbuf, gsem)
        dj = pltpu.async_copy(cols_hbm.at[pl.ds(boff * _U, _UPB * _U)],
                              colbuf, gsem)
        dk = pltpu.async_copy(vals_hbm.at[pl.ds(boff, _UPB), :], valbuf, gsem)
        di.wait()
        dj.wait()
        dk.wait()
        pltpu.async_copy(dloc.at[colbuf.at[pl.ds(0, _G * _U)]],
                         gsub.at[0], gsem)

        def _unit(u, carry):
            p = u % 2
            pn = (u + 1) % 2

            @pl.when(u >= 1)
            def _():
                pltpu.make_async_copy(dloc.at[pl.ds(0, _G * _U)],
                                      gsub.at[pn], ssem).wait()

            @pl.when(u + 1 < _SPB)
            def _():
                pltpu.async_copy(
                    dloc.at[colbuf.at[pl.ds((u + 1) * _G * _U, _G * _U)]],
                    gsub.at[pn], gsem)

            pltpu.make_async_copy(dloc.at[pl.ds(0, _G * _U)],
                                  gsub.at[p], gsem).wait()

            def _grp(g, carry2):
                g16 = (g % 8) * 16
                hrow = u * _G + g // 8
                r0 = (g // 8) * _U + g16
                val = valbuf[hrow, pl.ds(g16, 16)]
                for k in range(16):
                    r = r0 + k
                    v = val[k]
                    for q in range(_DH // 16):
                        gsub[p, r, pl.ds(q * 16, 16)] = \
                            gsub[p, r, pl.ds(q * 16, 16)] * v
                return carry2

            lax.fori_loop(0, _G * _U // 16, _grp, 0)

            for gg in range(_G):
                pltpu.async_copy(gsub.at[p, pl.ds(gg * _U, _U), :],
                                 acc.at[rowbuf.at[u * _G + gg]],
                                 ssem, add=True)
            return carry

        lax.fori_loop(0, _SPB, _unit, 0)
        pltpu.make_async_copy(dloc.at[pl.ds(0, _G * _U)],
                              gsub.at[(_SPB - 1) % 2], ssem).wait()

    plsc.subcore_barrier()
    start = jnp.minimum(sid * _WSLICE, _WLAST)
    pltpu.sync_copy(acc.at[pl.ds(start, _WSLICE), :],
                    out_hbm.at[cid, pl.ds(start, _WSLICE), :])


_spmm = pl.kernel(
    _spmm_body,
    out_type=jax.ShapeDtypeStruct((_NC, _N, _DH), jnp.float32),
    mesh=_mesh(),
    compiler_params=pltpu.CompilerParams(use_tc_tiling_on_sc=False),
    scratch_types=[
        pltpu.VMEM((_UPB, _U), jnp.int32),
        pltpu.VMEM((_UPB * _U,), jnp.int32),
        pltpu.VMEM((_UPB, _U), jnp.float32),
        pltpu.VMEM((2, _G * _U, _DH), jnp.float32),
        pltpu.VMEM_SHARED((_N, _DH), jnp.float32),
        pltpu.SemaphoreType.DMA,
        pltpu.SemaphoreType.DMA,
        pltpu.SemaphoreType.DMA,
    ],
)


def _edge_body(h_hbm, e_hbm, ac_hbm, feats_hbm,
               ebuf, g0, g1, fbuf, acbuf, gsem):
    cid = lax.axis_index("c")
    sid = lax.axis_index("s")
    w = sid * _NC + cid
    base = w * _EPP
    pltpu.sync_copy(ac_hbm, acbuf)
    nu = _EPP // 128
    descs = []
    for u in range(nu):
        descs.append(pltpu.async_copy(
            e_hbm.at[0, pl.ds(base + u * 128, 128)], ebuf.at[0, u], gsem))
        descs.append(pltpu.async_copy(
            e_hbm.at[1, pl.ds(base + u * 128, 128)], ebuf.at[1, u], gsem))
    for de in descs:
        de.wait()

    def _unit(u, carry):
        da = pltpu.async_copy(h_hbm.at[ebuf.at[0, u]], g0, gsem)
        db = pltpu.async_copy(h_hbm.at[ebuf.at[1, u]], g1, gsem)
        da.wait()
        db.wait()

        def _row(j, carry2):
            for q in range(4):
                a_q = acbuf[0, pl.ds(q * 16, 16)]
                c_q = acbuf[1, pl.ds(q * 16, 16)]
                fbuf[j, pl.ds(q * 16, 16)] = g0[j, pl.ds(q * 16, 16)] * a_q + c_q
                fbuf[j, pl.ds(64 + q * 16, 16)] = g1[j, pl.ds(q * 16, 16)] * a_q + c_q
            return carry2

        lax.fori_loop(0, 128, _row, 0)
        pltpu.sync_copy(fbuf, feats_hbm.at[pl.ds(base + u * 128, 128), :])
        return carry

    lax.fori_loop(0, nu, _unit, 0)


_edge = pl.kernel(
    _edge_body,
    out_type=jax.ShapeDtypeStruct((_EPPAD, 2 * _H), jnp.float32),
    mesh=_mesh(),
    compiler_params=pltpu.CompilerParams(use_tc_tiling_on_sc=False),
    scratch_types=[
        pltpu.VMEM((2, _EPP // 128, 128), jnp.int32),
        pltpu.VMEM((128, _H), jnp.float32),
        pltpu.VMEM((128, _H), jnp.float32),
        pltpu.VMEM((128, 2 * _H), jnp.float32),
        pltpu.VMEM((2, _H), jnp.float32),
        pltpu.SemaphoreType.DMA,
    ],
)


_BLK = 2000


def _sup_body(x_ref, b0_ref, w1_ref, o_ref):
    x = jnp.concatenate([x_ref[0], x_ref[1]], axis=1)
    x = jnp.maximum(x + b0_ref[...], 0.0)
    res = jnp.dot(x, w1_ref[...], preferred_element_type=jnp.float32)
    o_ref[0] = res[:, :_DH]
    o_ref[1] = res[:, _DH:]


def _sup(x, b0, W1):
    grid = _N // _BLK
    return pl.pallas_call(
        _sup_body,
        grid=(grid,),
        in_specs=[
            pl.BlockSpec((_NC, _BLK, _DH), lambda i: (0, i, 0)),
            pl.BlockSpec((1, _D), lambda i: (0, 0)),
            pl.BlockSpec((_D, _D), lambda i: (0, 0)),
        ],
        out_specs=pl.BlockSpec((_NC, _BLK, _DH), lambda i: (0, i, 0)),
        out_shape=jax.ShapeDtypeStruct((_NC, _N, _DH), jnp.float32),
    )(x, b0.reshape(1, _D), W1)


def _gru_body(x0_ref, x1_ref, x2_ref, wihT_ref, whhT_ref, bih_ref, bhh_ref,
              b1_ref, tm_ref, h_ref, st_ref):
    i = pl.program_id(0)
    h = jnp.zeros((_BLK, _H), jnp.float32)
    xs = (x0_ref, x1_ref, x2_ref)
    wihT = wihT_ref[...]
    whhT = whhT_ref[...]
    for t in range(_T):
        x = jnp.concatenate([xs[t][0], xs[t][1]], axis=1) + b1_ref[...]
        gi = jnp.dot(x, wihT, preferred_element_type=jnp.float32) + bih_ref[...]
        gh = jnp.dot(h, whhT, preferred_element_type=jnp.float32) + bhh_ref[...]
        r = jax.nn.sigmoid(gi[:, :_H] + gh[:, :_H])
        z = jax.nn.sigmoid(gi[:, _H:2 * _H] + gh[:, _H:2 * _H])
        n = jnp.tanh(gi[:, 2 * _H:] + r * gh[:, 2 * _H:])
        h_new = (1.0 - z) * n + z * h
        tm = tm_ref[0, t]
        h = tm * h_new + (1.0 - tm) * h
    h_ref[...] = h

    @pl.when(i == 0)
    def _():
        st_ref[...] = jnp.zeros_like(st_ref)

    st_ref[0:1, :] += jnp.sum(h, axis=0, keepdims=True)
    st_ref[1:2, :] += jnp.sum(h * h, axis=0, keepdims=True)


def _gru(x0, x1, x2, wihT, whhT, b_ih, b_hh, b1, tmask):
    grid = _N // _BLK
    return pl.pallas_call(
        _gru_body,
        grid=(grid,),
        in_specs=[
            pl.BlockSpec((_NC, _BLK, _DH), lambda i: (0, i, 0)),
            pl.BlockSpec((_NC, _BLK, _DH), lambda i: (0, i, 0)),
            pl.BlockSpec((_NC, _BLK, _DH), lambda i: (0, i, 0)),
            pl.BlockSpec((_D, 3 * _H), lambda i: (0, 0)),
            pl.BlockSpec((_H, 3 * _H), lambda i: (0, 0)),
            pl.BlockSpec((1, 3 * _H), lambda i: (0, 0)),
            pl.BlockSpec((1, 3 * _H), lambda i: (0, 0)),
            pl.BlockSpec((1, _D), lambda i: (0, 0)),
            pl.BlockSpec((1, _T), lambda i: (0, 0), memory_space=pltpu.SMEM),
        ],
        out_specs=[
            pl.BlockSpec((_BLK, _H), lambda i: (i, 0)),
            pl.BlockSpec((8, _H), lambda i: (0, 0)),
        ],
        out_shape=[
            jax.ShapeDtypeStruct((_N, _H), jnp.float32),
            jax.ShapeDtypeStruct((8, _H), jnp.float32),
        ],
    )(x0, x1, x2, wihT, whhT, b_ih.reshape(1, -1), b_hh.reshape(1, -1),
      b1.reshape(1, -1), tmask)


_MBLK = 2048


def _mlp_body(f_ref, wp1_ref, bp1_ref, wp2_ref, bp2_ref, o_ref):
    hmid = jnp.maximum(
        jnp.dot(f_ref[...], wp1_ref[...], preferred_element_type=jnp.float32)
        + bp1_ref[...], 0.0)
    lg = jnp.dot(hmid, wp2_ref[...], preferred_element_type=jnp.float32) \
        + bp2_ref[...]
    m = jnp.max(lg, axis=1, keepdims=True)
    e = jnp.exp(lg - m)
    o_ref[...] = (lg - m) - jnp.log(jnp.sum(e, axis=1, keepdims=True))


def _mlp(feats, Wp1, bp1, Wp2, bp2):
    grid = _EPPAD // _MBLK
    return pl.pallas_call(
        _mlp_body,
        grid=(grid,),
        in_specs=[
            pl.BlockSpec((_MBLK, _NHE), lambda i: (i, 0)),
            pl.BlockSpec((_NHE, _NHE), lambda i: (0, 0)),
            pl.BlockSpec((1, _NHE), lambda i: (0, 0)),
            pl.BlockSpec((_NHE, 2), lambda i: (0, 0)),
            pl.BlockSpec((1, 2), lambda i: (0, 0)),
        ],
        out_specs=pl.BlockSpec((_MBLK, 2), lambda i: (i, 0)),
        out_shape=jax.ShapeDtypeStruct((_EPPAD, 2), jnp.float32),
    )(feats, Wp1, bp1.reshape(1, -1), Wp2, bp2.reshape(1, -1))


def kernel(start_day, end_day, adj_indices, adj_values, edges,
           W0, b0, W1, b1, W_ih, W_hh, b_ih, b_hh, gamma, beta,
           Wp1, bp1, Wp2, bp2):
    adj_indices = adj_indices.astype(jnp.int32)
    edges = edges.astype(jnp.int32)
    pad = _EPAD - _E
    adi = jnp.pad(adj_indices, ((0, 0), (0, 0), (0, pad)))
    adv = jnp.pad(adj_values, ((0, 0), (0, pad)))
    W0s = jnp.stack([W0[:, :_DH], W0[:, _DH:]])

    outs = []
    for i in range(_T):
        t = start_day + i
        idx_t = lax.dynamic_index_in_dim(adi, t, 0, keepdims=False)
        val_t = lax.dynamic_index_in_dim(adv, t, 0, keepdims=False)
        r2 = idx_t[0].reshape(_EU, _U)
        c2 = idx_t[1]
        v2 = val_t.reshape(_EU, _U)
        x1 = _spmm(r2, c2, v2, W0s)
        sup = _sup(x1, b0, W1)
        outs.append(_spmm(r2, c2, v2, sup))

    tmask = ((start_day + jnp.arange(_T)) <= end_day) \
        .astype(jnp.float32).reshape(1, _T)
    h, stats = _gru(outs[0], outs[1], outs[2], W_ih.T, W_hh.T,
                    b_ih, b_hh, b1, tmask)
    mean = stats[0, :] / _N
    var = stats[1, :] / _N - mean * mean
    a = gamma * lax.rsqrt(var + 1e-5)
    c = beta - a * mean

    epad = jnp.pad(edges, ((0, 0), (0, _EPPAD - _EP)))
    feats = _edge(h, epad, jnp.stack([a, c]))
    return _mlp(feats, Wp1, bp1, Wp2, bp2)[:_EP]


# speedup vs baseline: 8.9953x; 1.1945x over previous
import functools

import jax
import jax.numpy as jnp
from jax import lax
from jax.experimental import pallas as pl
from jax.experimental.pallas import tpu as pltpu
from jax.experimental.pallas import tpu_sc as plsc

_N = 50000
_T = 3
_E = 800000
_D = 64
_H = 64
_NHE = 128
_EP = 100000

_NC = 2
_NS = 16
_DH = _D // _NC
_U = 128
_UPB = 23
_NBLK = 17
_UPT = _UPB * _NBLK
_EPT = _UPT * _U
_EPAD = _NS * _EPT
_EU = _EPAD // _U
_WSLICE = 3136
_WLAST = _N - _WSLICE

_EPP = 3200
_EPPAD = _EPP * _NC * _NS

_mesh = functools.partial(
    plsc.VectorSubcoreMesh, core_axis_name="c", subcore_axis_name="s",
    num_cores=_NC, num_subcores=_NS)


def _spmm_body(rows_hbm, cols_hbm, vals_hbm, d_hbm, out_hbm,
               rowbuf, colbuf, valbuf, gsub, acc, gsem, ssem, isem, zsem):
    cid = lax.axis_index("c")
    sid = lax.axis_index("s")
    dloc = d_hbm.at[cid]

    zero = jnp.zeros((16,), jnp.float32)

    def _zrow(i, carry):
        for q in range(_DH // 16):
            gsub[0, i, pl.ds(q * 16, 16)] = zero
        return carry

    lax.fori_loop(0, _U, _zrow, 0)
    zstart = jnp.minimum(sid * _WSLICE, _WLAST)
    zdescs = [
        pltpu.async_copy(gsub.at[0],
                         acc.at[pl.ds(zstart + z * _U, _U), :], zsem)
        for z in range(_WSLICE // _U)
    ]
    zdescs.append(pltpu.async_copy(
        gsub.at[0, pl.ds(0, _WSLICE % _U), :],
        acc.at[pl.ds(zstart + (_WSLICE // _U) * _U, _WSLICE % _U), :], zsem))
    for de in zdescs:
        de.wait()
    plsc.subcore_barrier()

    ubase = sid * _UPT

    def _stage(blk):
        par = blk % 2
        boff = ubase + blk * _UPB
        pltpu.async_copy(rows_hbm.at[pl.ds(boff, _UPB), :],
                         rowbuf.at[par], isem)
        pltpu.async_copy(cols_hbm.at[pl.ds(boff * _U, _UPB * _U)],
                         colbuf.at[par], isem)
        pltpu.async_copy(vals_hbm.at[pl.ds(boff, _UPB), :],
                         valbuf.at[par], isem)

    def _drain_idx():
        pltpu.make_async_copy(rows_hbm.at[pl.ds(0, _UPB), :],
                              rowbuf.at[0], isem).wait()
        pltpu.make_async_copy(cols_hbm.at[pl.ds(0, _UPB * _U)],
                              colbuf.at[0], isem).wait()
        pltpu.make_async_copy(vals_hbm.at[pl.ds(0, _UPB), :],
                              valbuf.at[0], isem).wait()

    def _fire_gather(u, dst_p):
        blk = u // _UPB
        pos = u % _UPB
        pltpu.async_copy(
            dloc.at[colbuf.at[blk % 2, pl.ds(pos * _U, _U)]],
            gsub.at[dst_p], gsem)

    _stage(0)
    _stage(1)
    _drain_idx()
    _fire_gather(0, 0)
    _fire_gather(1, 1)

    def _step(u, carry):
        p = u % 3
        blk = u // _UPB
        pos = u % _UPB

        @pl.when(u >= 1)
        def _():
            pltpu.make_async_copy(dloc.at[pl.ds(0, _U)],
                                  gsub.at[0], ssem).wait()

        @pl.when((pos == 1) & (blk >= 1) & (blk < _NBLK - 1))
        def _():
            _stage(blk + 1)

        @pl.when((pos == _UPB - 3) & (blk < _NBLK - 1))
        def _():
            _drain_idx()

        @pl.when(u + 2 < _UPT)
        def _():
            _fire_gather(u + 2, (u + 2) % 3)

        pltpu.make_async_copy(dloc.at[pl.ds(0, _U)], gsub.at[0], gsem).wait()

        def _grp(g, carry2):
            g16 = g * 16
            val = valbuf[blk % 2, pos, pl.ds(g16, 16)]
            for k in range(16):
                r = g16 + k
                v = val[k]
                for q in range(_DH // 16):
                    gsub[p, r, pl.ds(q * 16, 16)] = \
                        gsub[p, r, pl.ds(q * 16, 16)] * v
            return carry2

        lax.fori_loop(0, _U // 16, _grp, 0)

        pltpu.async_copy(gsub.at[p], acc.at[rowbuf.at[blk % 2, pos]],
                         ssem, add=True)
        return carry

    lax.fori_loop(0, _UPT, _step, 0)
    pltpu.make_async_copy(dloc.at[pl.ds(0, _U)], gsub.at[0], ssem).wait()

    plsc.subcore_barrier()
    start = jnp.minimum(sid * _WSLICE, _WLAST)
    pltpu.sync_copy(acc.at[pl.ds(start, _WSLICE), :],
                    out_hbm.at[cid, pl.ds(start, _WSLICE), :])


_spmm = pl.kernel(
    _spmm_body,
    out_type=jax.ShapeDtypeStruct((_NC, _N, _DH), jnp.float32),
    mesh=_mesh(),
    compiler_params=pltpu.CompilerParams(use_tc_tiling_on_sc=False),
    scratch_types=[
        pltpu.VMEM((2, _UPB, _U), jnp.int32),
        pltpu.VMEM((2, _UPB * _U), jnp.int32),
        pltpu.VMEM((2, _UPB, _U), jnp.float32),
        pltpu.VMEM((3, _U, _DH), jnp.float32),
        pltpu.VMEM_SHARED((_N, _DH), jnp.float32),
        pltpu.SemaphoreType.DMA,
        pltpu.SemaphoreType.DMA,
        pltpu.SemaphoreType.DMA,
        pltpu.SemaphoreType.DMA,
    ],
)


_ENU = _EPP // 128


def _edge_body(h_hbm, e_hbm, ac_hbm, feats_hbm,
               ebuf, gg, fbuf, acbuf, gsem, wsem):
    cid = lax.axis_index("c")
    sid = lax.axis_index("s")
    w = sid * _NC + cid
    base = w * _EPP
    pltpu.sync_copy(ac_hbm, acbuf)
    d0 = pltpu.async_copy(e_hbm.at[0, pl.ds(base, _EPP)], ebuf.at[0], gsem)
    d1 = pltpu.async_copy(e_hbm.at[1, pl.ds(base, _EPP)], ebuf.at[1], gsem)
    d0.wait()
    d1.wait()

    av = [acbuf[0, pl.ds(q * 16, 16)] for q in range(4)]
    cv = [acbuf[1, pl.ds(q * 16, 16)] for q in range(4)]

    def _fire(u, p):
        pltpu.async_copy(h_hbm.at[ebuf.at[0, pl.ds(u * 128, 128)]],
                         gg.at[p, 0], gsem)
        pltpu.async_copy(h_hbm.at[ebuf.at[1, pl.ds(u * 128, 128)]],
                         gg.at[p, 1], gsem)

    _fire(0, 0)

    def _unit(u, carry):
        p = u % 2
        pn = (u + 1) % 2

        @pl.when(u >= 2)
        def _():
            pltpu.make_async_copy(feats_hbm.at[pl.ds(0, 128), :],
                                  fbuf.at[p], wsem).wait()

        @pl.when(u + 1 < _ENU)
        def _():
            _fire(u + 1, pn)

        pltpu.make_async_copy(h_hbm.at[pl.ds(0, 128)], gg.at[p, 0], gsem).wait()
        pltpu.make_async_copy(h_hbm.at[pl.ds(0, 128)], gg.at[p, 1], gsem).wait()

        def _row(j, carry2):
            for q in range(4):
                fbuf[p, j, pl.ds(q * 16, 16)] = \
                    gg[p, 0, j, pl.ds(q * 16, 16)] * av[q] + cv[q]
                fbuf[p, j, pl.ds(64 + q * 16, 16)] = \
                    gg[p, 1, j, pl.ds(q * 16, 16)] * av[q] + cv[q]
            return carry2

        lax.fori_loop(0, 128, _row, 0)
        pltpu.async_copy(fbuf.at[p],
                         feats_hbm.at[pl.ds(base + u * 128, 128), :], wsem)
        return carry

    lax.fori_loop(0, _ENU, _unit, 0)
    for _ in range(2):
        pltpu.make_async_copy(feats_hbm.at[pl.ds(0, 128), :],
                              fbuf.at[0], wsem).wait()


_edge = pl.kernel(
    _edge_body,
    out_type=jax.ShapeDtypeStruct((_EPPAD, 2 * _H), jnp.float32),
    mesh=_mesh(),
    compiler_params=pltpu.CompilerParams(use_tc_tiling_on_sc=False),
    scratch_types=[
        pltpu.VMEM((2, _EPP), jnp.int32),
        pltpu.VMEM((2, 2, 128, _H), jnp.float32),
        pltpu.VMEM((2, 128, 2 * _H), jnp.float32),
        pltpu.VMEM((2, _H), jnp.float32),
        pltpu.SemaphoreType.DMA,
        pltpu.SemaphoreType.DMA,
    ],
)


_BLK = 2000


def _sup_body(x_ref, b0_ref, w1_ref, o_ref):
    x = jnp.concatenate([x_ref[0], x_ref[1]], axis=1)
    x = jnp.maximum(x + b0_ref[...], 0.0)
    res = jnp.dot(x, w1_ref[...], preferred_element_type=jnp.float32)
    o_ref[0] = res[:, :_DH]
    o_ref[1] = res[:, _DH:]


def _sup(x, b0, W1):
    grid = _N // _BLK
    return pl.pallas_call(
        _sup_body,
        grid=(grid,),
        in_specs=[
            pl.BlockSpec((_NC, _BLK, _DH), lambda i: (0, i, 0)),
            pl.BlockSpec((1, _D), lambda i: (0, 0)),
            pl.BlockSpec((_D, _D), lambda i: (0, 0)),
        ],
        out_specs=pl.BlockSpec((_NC, _BLK, _DH), lambda i: (0, i, 0)),
        out_shape=jax.ShapeDtypeStruct((_NC, _N, _DH), jnp.float32),
    )(x, b0.reshape(1, _D), W1)


def _gru_body(x0_ref, x1_ref, x2_ref, wihT_ref, whhT_ref, bih_ref, bhh_ref,
              b1_ref, tm_ref, h_ref, st_ref):
    i = pl.program_id(0)
    h = jnp.zeros((_BLK, _H), jnp.float32)
    xs = (x0_ref, x1_ref, x2_ref)
    wihT = wihT_ref[...]
    whhT = whhT_ref[...]
    for t in range(_T):
        x = jnp.concatenate([xs[t][0], xs[t][1]], axis=1) + b1_ref[...]
        gi = jnp.dot(x, wihT, preferred_element_type=jnp.float32) + bih_ref[...]
        gh = jnp.dot(h, whhT, preferred_element_type=jnp.float32) + bhh_ref[...]
        r = jax.nn.sigmoid(gi[:, :_H] + gh[:, :_H])
        z = jax.nn.sigmoid(gi[:, _H:2 * _H] + gh[:, _H:2 * _H])
        n = jnp.tanh(gi[:, 2 * _H:] + r * gh[:, 2 * _H:])
        h_new = (1.0 - z) * n + z * h
        tm = tm_ref[0, t]
        h = tm * h_new + (1.0 - tm) * h
    h_ref[...] = h

    @pl.when(i == 0)
    def _():
        st_ref[...] = jnp.zeros_like(st_ref)

    st_ref[0:1, :] += jnp.sum(h, axis=0, keepdims=True)
    st_ref[1:2, :] += jnp.sum(h * h, axis=0, keepdims=True)


def _gru(x0, x1, x2, wihT, whhT, b_ih, b_hh, b1, tmask):
    grid = _N // _BLK
    return pl.pallas_call(
        _gru_body,
        grid=(grid,),
        in_specs=[
            pl.BlockSpec((_NC, _BLK, _DH), lambda i: (0, i, 0)),
            pl.BlockSpec((_NC, _BLK, _DH), lambda i: (0, i, 0)),
            pl.BlockSpec((_NC, _BLK, _DH), lambda i: (0, i, 0)),
            pl.BlockSpec((_D, 3 * _H), lambda i: (0, 0)),
            pl.BlockSpec((_H, 3 * _H), lambda i: (0, 0)),
            pl.BlockSpec((1, 3 * _H), lambda i: (0, 0)),
            pl.BlockSpec((1, 3 * _H), lambda i: (0, 0)),
            pl.BlockSpec((1, _D), lambda i: (0, 0)),
            pl.BlockSpec((1, _T), lambda i: (0, 0), memory_space=pltpu.SMEM),
        ],
        out_specs=[
            pl.BlockSpec((_BLK, _H), lambda i: (i, 0)),
            pl.BlockSpec((8, _H), lambda i: (0, 0)),
        ],
        out_shape=[
            jax.ShapeDtypeStruct((_N, _H), jnp.float32),
            jax.ShapeDtypeStruct((8, _H), jnp.float32),
        ],
    )(x0, x1, x2, wihT, whhT, b_ih.reshape(1, -1), b_hh.reshape(1, -1),
      b1.reshape(1, -1), tmask)


_MBLK = 2048


def _mlp_body(f_ref, wp1_ref, bp1_ref, wp2_ref, bp2_ref, o_ref):
    hmid = jnp.maximum(
        jnp.dot(f_ref[...], wp1_ref[...], preferred_element_type=jnp.float32)
        + bp1_ref[...], 0.0)
    lg = jnp.dot(hmid, wp2_ref[...], preferred_element_type=jnp.float32) \
        + bp2_ref[...]
    m = jnp.max(lg, axis=1, keepdims=True)
    e = jnp.exp(lg - m)
    o_ref[...] = (lg - m) - jnp.log(jnp.sum(e, axis=1, keepdims=True))


def _mlp(feats, Wp1, bp1, Wp2, bp2):
    grid = _EPPAD // _MBLK
    return pl.pallas_call(
        _mlp_body,
        grid=(grid,),
        in_specs=[
            pl.BlockSpec((_MBLK, _NHE), lambda i: (i, 0)),
            pl.BlockSpec((_NHE, _NHE), lambda i: (0, 0)),
            pl.BlockSpec((1, _NHE), lambda i: (0, 0)),
            pl.BlockSpec((_NHE, 2), lambda i: (0, 0)),
            pl.BlockSpec((1, 2), lambda i: (0, 0)),
        ],
        out_specs=pl.BlockSpec((_MBLK, 2), lambda i: (i, 0)),
        out_shape=jax.ShapeDtypeStruct((_EPPAD, 2), jnp.float32),
    )(feats, Wp1, bp1.reshape(1, -1), Wp2, bp2.reshape(1, -1))


def kernel(start_day, end_day, adj_indices, adj_values, edges,
           W0, b0, W1, b1, W_ih, W_hh, b_ih, b_hh, gamma, beta,
           Wp1, bp1, Wp2, bp2):
    adj_indices = adj_indices.astype(jnp.int32)
    edges = edges.astype(jnp.int32)
    pad = _EPAD - _E
    adi = jnp.pad(adj_indices, ((0, 0), (0, 0), (0, pad)))
    adv = jnp.pad(adj_values, ((0, 0), (0, pad)))
    W0s = jnp.stack([W0[:, :_DH], W0[:, _DH:]])

    outs = []
    for i in range(_T):
        t = start_day + i
        idx_t = lax.dynamic_index_in_dim(adi, t, 0, keepdims=False)
        val_t = lax.dynamic_index_in_dim(adv, t, 0, keepdims=False)
        r2 = idx_t[0].reshape(_EU, _U)
        c2 = idx_t[1]
        v2 = val_t.reshape(_EU, _U)
        x1 = _spmm(r2, c2, v2, W0s)
        sup = _sup(x1, b0, W1)
        outs.append(_spmm(r2, c2, v2, sup))

    tmask = ((start_day + jnp.arange(_T)) <= end_day) \
        .astype(jnp.float32).reshape(1, _T)
    h, stats = _gru(outs[0], outs[1], outs[2], W_ih.T, W_hh.T,
                    b_ih, b_hh, b1, tmask)
    mean = stats[0, :] / _N
    var = stats[1, :] / _N - mean * mean
    a = gamma * lax.rsqrt(var + 1e-5)
    c = beta - a * mean

    epad = jnp.pad(edges, ((0, 0), (0, _EPPAD - _EP)))
    feats = _edge(h, epad, jnp.stack([a, c]))
    return _mlp(feats, Wp1, bp1, Wp2, bp2)[:_EP]


# speedup vs baseline: 9.3537x; 1.0399x over previous
import functools

import jax
import jax.numpy as jnp
from jax import lax
from jax.experimental import pallas as pl
from jax.experimental.pallas import tpu as pltpu
from jax.experimental.pallas import tpu_sc as plsc

_N = 50000
_T = 3
_E = 800000
_D = 64
_H = 64
_NHE = 128
_EP = 100000

_NC = 2
_NS = 16
_DH = _D // _NC
_U = 128
_UPB = 17
_NBLK = 23
_UPT = _UPB * _NBLK
_EPT = _UPT * _U
_EPAD = _NS * _EPT
_EU = _EPAD // _U
_WSLICE = 3136
_WLAST = _N - _WSLICE

_EPP = 3200
_EPPAD = _EPP * _NC * _NS

_mesh = functools.partial(
    plsc.VectorSubcoreMesh, core_axis_name="c", subcore_axis_name="s",
    num_cores=_NC, num_subcores=_NS)


def _spmm_body(rows_hbm, cols_hbm, vals_hbm, d_hbm, out_hbm,
               rowbuf, colbuf, valbuf, gsub, acc, gsem, ssem, isem, zsem):
    cid = lax.axis_index("c")
    sid = lax.axis_index("s")
    dloc = d_hbm.at[cid]

    zero = jnp.zeros((16,), jnp.float32)

    def _zrow(i, carry):
        for q in range(_DH // 16):
            gsub[0, i, pl.ds(q * 16, 16)] = zero
        return carry

    lax.fori_loop(0, _U, _zrow, 0)
    zstart = jnp.minimum(sid * _WSLICE, _WLAST)
    zdescs = [
        pltpu.async_copy(gsub.at[0],
                         acc.at[pl.ds(zstart + z * _U, _U), :], zsem)
        for z in range(_WSLICE // _U)
    ]
    zdescs.append(pltpu.async_copy(
        gsub.at[0, pl.ds(0, _WSLICE % _U), :],
        acc.at[pl.ds(zstart + (_WSLICE // _U) * _U, _WSLICE % _U), :], zsem))
    for de in zdescs:
        de.wait()
    plsc.subcore_barrier()

    ubase = sid * _UPT

    def _stage(blk):
        par = blk % 2
        boff = ubase + blk * _UPB
        pltpu.async_copy(rows_hbm.at[pl.ds(boff, _UPB), :],
                         rowbuf.at[par], isem)
        pltpu.async_copy(cols_hbm.at[pl.ds(boff * _U, _UPB * _U)],
                         colbuf.at[par], isem)
        pltpu.async_copy(vals_hbm.at[pl.ds(boff, _UPB), :],
                         valbuf.at[par], isem)

    def _drain_idx():
        pltpu.make_async_copy(rows_hbm.at[pl.ds(0, _UPB), :],
                              rowbuf.at[0], isem).wait()
        pltpu.make_async_copy(cols_hbm.at[pl.ds(0, _UPB * _U)],
                              colbuf.at[0], isem).wait()
        pltpu.make_async_copy(vals_hbm.at[pl.ds(0, _UPB), :],
                              valbuf.at[0], isem).wait()

    def _fire_gather(u, dst_p):
        blk = u // _UPB
        pos = u % _UPB
        pltpu.async_copy(
            dloc.at[colbuf.at[blk % 2, pl.ds(pos * _U, _U)]],
            gsub.at[dst_p], gsem)

    _stage(0)
    _stage(1)
    _drain_idx()
    _fire_gather(0, 0)
    _fire_gather(1, 1)
    _fire_gather(2, 2)

    def _step(u, carry):
        p = u % 4
        blk = u // _UPB
        pos = u % _UPB

        @pl.when(u >= 1)
        def _():
            pltpu.make_async_copy(dloc.at[pl.ds(0, _U)],
                                  gsub.at[0], ssem).wait()

        @pl.when((pos == 1) & (blk >= 1) & (blk < _NBLK - 1))
        def _():
            _stage(blk + 1)

        @pl.when((pos == _UPB - 4) & (blk < _NBLK - 1))
        def _():
            _drain_idx()

        @pl.when(u + 3 < _UPT)
        def _():
            _fire_gather(u + 3, (u + 3) % 4)

        pltpu.make_async_copy(dloc.at[pl.ds(0, _U)], gsub.at[0], gsem).wait()

        def _grp(g, carry2):
            g16 = g * 16
            val = valbuf[blk % 2, pos, pl.ds(g16, 16)]
            for k in range(16):
                r = g16 + k
                v = val[k]
                for q in range(_DH // 16):
                    gsub[p, r, pl.ds(q * 16, 16)] = \
                        gsub[p, r, pl.ds(q * 16, 16)] * v
            return carry2

        lax.fori_loop(0, _U // 16, _grp, 0)

        pltpu.async_copy(gsub.at[p], acc.at[rowbuf.at[blk % 2, pos]],
                         ssem, add=True)
        return carry

    lax.fori_loop(0, _UPT, _step, 0)
    pltpu.make_async_copy(dloc.at[pl.ds(0, _U)], gsub.at[0], ssem).wait()

    plsc.subcore_barrier()
    start = jnp.minimum(sid * _WSLICE, _WLAST)
    pltpu.sync_copy(acc.at[pl.ds(start, _WSLICE), :],
                    out_hbm.at[cid, pl.ds(start, _WSLICE), :])


_spmm = pl.kernel(
    _spmm_body,
    out_type=jax.ShapeDtypeStruct((_NC, _N, _DH), jnp.float32),
    mesh=_mesh(),
    compiler_params=pltpu.CompilerParams(use_tc_tiling_on_sc=False),
    scratch_types=[
        pltpu.VMEM((2, _UPB, _U), jnp.int32),
        pltpu.VMEM((2, _UPB * _U), jnp.int32),
        pltpu.VMEM((2, _UPB, _U), jnp.float32),
        pltpu.VMEM((4, _U, _DH), jnp.float32),
        pltpu.VMEM_SHARED((_N, _DH), jnp.float32),
        pltpu.SemaphoreType.DMA,
        pltpu.SemaphoreType.DMA,
        pltpu.SemaphoreType.DMA,
        pltpu.SemaphoreType.DMA,
    ],
)


_ENU = _EPP // 128


def _edge_body(h_hbm, e_hbm, ac_hbm, feats_hbm,
               ebuf, gg, fbuf, acbuf, gsem, wsem):
    cid = lax.axis_index("c")
    sid = lax.axis_index("s")
    w = sid * _NC + cid
    base = w * _EPP
    pltpu.sync_copy(ac_hbm, acbuf)
    d0 = pltpu.async_copy(e_hbm.at[0, pl.ds(base, _EPP)], ebuf.at[0], gsem)
    d1 = pltpu.async_copy(e_hbm.at[1, pl.ds(base, _EPP)], ebuf.at[1], gsem)
    d0.wait()
    d1.wait()

    av = [acbuf[0, pl.ds(q * 16, 16)] for q in range(4)]
    cv = [acbuf[1, pl.ds(q * 16, 16)] for q in range(4)]

    def _fire(u, p):
        pltpu.async_copy(h_hbm.at[ebuf.at[0, pl.ds(u * 128, 128)]],
                         gg.at[p, 0], gsem)
        pltpu.async_copy(h_hbm.at[ebuf.at[1, pl.ds(u * 128, 128)]],
                         gg.at[p, 1], gsem)

    _fire(0, 0)

    def _unit(u, carry):
        p = u % 2
        pn = (u + 1) % 2

        @pl.when(u >= 2)
        def _():
            pltpu.make_async_copy(feats_hbm.at[pl.ds(0, 128), :],
                                  fbuf.at[p], wsem).wait()

        @pl.when(u + 1 < _ENU)
        def _():
            _fire(u + 1, pn)

        pltpu.make_async_copy(h_hbm.at[pl.ds(0, 128)], gg.at[p, 0], gsem).wait()
        pltpu.make_async_copy(h_hbm.at[pl.ds(0, 128)], gg.at[p, 1], gsem).wait()

        def _row(j, carry2):
            for q in range(4):
                fbuf[p, j, pl.ds(q * 16, 16)] = \
                    gg[p, 0, j, pl.ds(q * 16, 16)] * av[q] + cv[q]
                fbuf[p, j, pl.ds(64 + q * 16, 16)] = \
                    gg[p, 1, j, pl.ds(q * 16, 16)] * av[q] + cv[q]
            return carry2

        lax.fori_loop(0, 128, _row, 0)
        pltpu.async_copy(fbuf.at[p],
                         feats_hbm.at[pl.ds(base + u * 128, 128), :], wsem)
        return carry

    lax.fori_loop(0, _ENU, _unit, 0)
    for _ in range(2):
        pltpu.make_async_copy(feats_hbm.at[pl.ds(0, 128), :],
                              fbuf.at[0], wsem).wait()


_edge = pl.kernel(
    _edge_body,
    out_type=jax.ShapeDtypeStruct((_EPPAD, 2 * _H), jnp.float32),
    mesh=_mesh(),
    compiler_params=pltpu.CompilerParams(use_tc_tiling_on_sc=False),
    scratch_types=[
        pltpu.VMEM((2, _EPP), jnp.int32),
        pltpu.VMEM((2, 2, 128, _H), jnp.float32),
        pltpu.VMEM((2, 128, 2 * _H), jnp.float32),
        pltpu.VMEM((2, _H), jnp.float32),
        pltpu.SemaphoreType.DMA,
        pltpu.SemaphoreType.DMA,
    ],
)


_BLK = 2000


def _sup_body(x_ref, b0_ref, w1_ref, o_ref):
    x = jnp.concatenate([x_ref[0], x_ref[1]], axis=1)
    x = jnp.maximum(x + b0_ref[...], 0.0)
    res = jnp.dot(x, w1_ref[...], preferred_element_type=jnp.float32)
    o_ref[0] = res[:, :_DH]
    o_ref[1] = res[:, _DH:]


def _sup(x, b0, W1):
    grid = _N // _BLK
    return pl.pallas_call(
        _sup_body,
        grid=(grid,),
        in_specs=[
            pl.BlockSpec((_NC, _BLK, _DH), lambda i: (0, i, 0)),
            pl.BlockSpec((1, _D), lambda i: (0, 0)),
            pl.BlockSpec((_D, _D), lambda i: (0, 0)),
        ],
        out_specs=pl.BlockSpec((_NC, _BLK, _DH), lambda i: (0, i, 0)),
        out_shape=jax.ShapeDtypeStruct((_NC, _N, _DH), jnp.float32),
    )(x, b0.reshape(1, _D), W1)


def _gru_body(x0_ref, x1_ref, x2_ref, wihT_ref, whhT_ref, bih_ref, bhh_ref,
              b1_ref, tm_ref, h_ref, st_ref):
    i = pl.program_id(0)
    h = jnp.zeros((_BLK, _H), jnp.float32)
    xs = (x0_ref, x1_ref, x2_ref)
    wihT = wihT_ref[...]
    whhT = whhT_ref[...]
    for t in range(_T):
        x = jnp.concatenate([xs[t][0], xs[t][1]], axis=1) + b1_ref[...]
        gi = jnp.dot(x, wihT, preferred_element_type=jnp.float32) + bih_ref[...]
        gh = jnp.dot(h, whhT, preferred_element_type=jnp.float32) + bhh_ref[...]
        r = jax.nn.sigmoid(gi[:, :_H] + gh[:, :_H])
        z = jax.nn.sigmoid(gi[:, _H:2 * _H] + gh[:, _H:2 * _H])
        n = jnp.tanh(gi[:, 2 * _H:] + r * gh[:, 2 * _H:])
        h_new = (1.0 - z) * n + z * h
        tm = tm_ref[0, t]
        h = tm * h_new + (1.0 - tm) * h
    h_ref[...] = h

    @pl.when(i == 0)
    def _():
        st_ref[...] = jnp.zeros_like(st_ref)

    st_ref[0:1, :] += jnp.sum(h, axis=0, keepdims=True)
    st_ref[1:2, :] += jnp.sum(h * h, axis=0, keepdims=True)


def _gru(x0, x1, x2, wihT, whhT, b_ih, b_hh, b1, tmask):
    grid = _N // _BLK
    return pl.pallas_call(
        _gru_body,
        grid=(grid,),
        in_specs=[
            pl.BlockSpec((_NC, _BLK, _DH), lambda i: (0, i, 0)),
            pl.BlockSpec((_NC, _BLK, _DH), lambda i: (0, i, 0)),
            pl.BlockSpec((_NC, _BLK, _DH), lambda i: (0, i, 0)),
            pl.BlockSpec((_D, 3 * _H), lambda i: (0, 0)),
            pl.BlockSpec((_H, 3 * _H), lambda i: (0, 0)),
            pl.BlockSpec((1, 3 * _H), lambda i: (0, 0)),
            pl.BlockSpec((1, 3 * _H), lambda i: (0, 0)),
            pl.BlockSpec((1, _D), lambda i: (0, 0)),
            pl.BlockSpec((1, _T), lambda i: (0, 0), memory_space=pltpu.SMEM),
        ],
        out_specs=[
            pl.BlockSpec((_BLK, _H), lambda i: (i, 0)),
            pl.BlockSpec((8, _H), lambda i: (0, 0)),
        ],
        out_shape=[
            jax.ShapeDtypeStruct((_N, _H), jnp.float32),
            jax.ShapeDtypeStruct((8, _H), jnp.float32),
        ],
    )(x0, x1, x2, wihT, whhT, b_ih.reshape(1, -1), b_hh.reshape(1, -1),
      b1.reshape(1, -1), tmask)


_MBLK = 2048


def _mlp_body(f_ref, wp1_ref, bp1_ref, wp2_ref, bp2_ref, o_ref):
    hmid = jnp.maximum(
        jnp.dot(f_ref[...], wp1_ref[...], preferred_element_type=jnp.float32)
        + bp1_ref[...], 0.0)
    lg = jnp.dot(hmid, wp2_ref[...], preferred_element_type=jnp.float32) \
        + bp2_ref[...]
    m = jnp.max(lg, axis=1, keepdims=True)
    e = jnp.exp(lg - m)
    o_ref[...] = (lg - m) - jnp.log(jnp.sum(e, axis=1, keepdims=True))


def _mlp(feats, Wp1, bp1, Wp2, bp2):
    grid = _EPPAD // _MBLK
    return pl.pallas_call(
        _mlp_body,
        grid=(grid,),
        in_specs=[
            pl.BlockSpec((_MBLK, _NHE), lambda i: (i, 0)),
            pl.BlockSpec((_NHE, _NHE), lambda i: (0, 0)),
            pl.BlockSpec((1, _NHE), lambda i: (0, 0)),
            pl.BlockSpec((_NHE, 2), lambda i: (0, 0)),
            pl.BlockSpec((1, 2), lambda i: (0, 0)),
        ],
        out_specs=pl.BlockSpec((_MBLK, 2), lambda i: (i, 0)),
        out_shape=jax.ShapeDtypeStruct((_EPPAD, 2), jnp.float32),
    )(feats, Wp1, bp1.reshape(1, -1), Wp2, bp2.reshape(1, -1))


def kernel(start_day, end_day, adj_indices, adj_values, edges,
           W0, b0, W1, b1, W_ih, W_hh, b_ih, b_hh, gamma, beta,
           Wp1, bp1, Wp2, bp2):
    adj_indices = adj_indices.astype(jnp.int32)
    edges = edges.astype(jnp.int32)
    pad = _EPAD - _E
    adi = jnp.pad(adj_indices, ((0, 0), (0, 0), (0, pad)))
    adv = jnp.pad(adj_values, ((0, 0), (0, pad)))
    W0s = jnp.stack([W0[:, :_DH], W0[:, _DH:]])

    outs = []
    for i in range(_T):
        t = start_day + i
        idx_t = lax.dynamic_index_in_dim(adi, t, 0, keepdims=False)
        val_t = lax.dynamic_index_in_dim(adv, t, 0, keepdims=False)
        r2 = idx_t[0].reshape(_EU, _U)
        c2 = idx_t[1]
        v2 = val_t.reshape(_EU, _U)
        x1 = _spmm(r2, c2, v2, W0s)
        sup = _sup(x1, b0, W1)
        outs.append(_spmm(r2, c2, v2, sup))

    tmask = ((start_day + jnp.arange(_T)) <= end_day) \
        .astype(jnp.float32).reshape(1, _T)
    h, stats = _gru(outs[0], outs[1], outs[2], W_ih.T, W_hh.T,
                    b_ih, b_hh, b1, tmask)
    mean = stats[0, :] / _N
    var = stats[1, :] / _N - mean * mean
    a = gamma * lax.rsqrt(var + 1e-5)
    c = beta - a * mean

    epad = jnp.pad(edges, ((0, 0), (0, _EPPAD - _EP)))
    feats = _edge(h, epad, jnp.stack([a, c]))
    return _mlp(feats, Wp1, bp1, Wp2, bp2)[:_EP]
